# baseline (speedup 1.0000x reference)
import math

import jax
import jax.numpy as jnp
from jax.experimental import pallas as pl
from jax.experimental.pallas import tpu as pltpu
from jax.experimental.pallas import tpu_sc as plsc

VOCAB = 1000000
EMBED = 64
BATCH = 16384
HIST = 50
B = BATCH * HIST

WINDOW = 128
LANES = 16
SCALE = math.sqrt(EMBED)

_mesh = plsc.VectorSubcoreMesh(core_axis_name="core", subcore_axis_name="subcore")


def _gather_scale(table, idx_flat):
    idx2d = idx_flat.reshape(1, B)

    @pl.kernel(
        out_type=jax.ShapeDtypeStruct((B, EMBED), jnp.float32),
        mesh=_mesh,
        scratch_types=[pltpu.VMEM((WINDOW, EMBED), jnp.float32)],
        compiler_params=pltpu.CompilerParams(use_tc_tiling_on_sc=False),
    )
    def kern(table_hbm, i_hbm, o_hbm, rows_v):
        def body(i_vmem, o_vmem):
            pltpu.sync_copy(table_hbm.at[i_vmem.at[0]], rows_v)

            @pl.loop(0, WINDOW)
            def _(r):
                @pl.loop(0, EMBED, step=LANES)
                def _(c):
                    o_vmem[r, pl.ds(c, LANES)] = (
                        rows_v[r, pl.ds(c, LANES)] * SCALE
                    )

        pltpu.emit_pipeline(
            body,
            grid=(B // WINDOW,),
            in_specs=[pl.BlockSpec((1, WINDOW), index_map=lambda i: (0, i))],
            out_specs=[pl.BlockSpec((WINDOW, EMBED), index_map=lambda i: (i, 0))],
            core_axis_name=("core", "subcore"),
            dimension_semantics=(pltpu.PARALLEL,),
        )(i_hbm, o_hbm)

    return kern(table, idx2d)


@jax.jit
def kernel(x, table):
    idx_flat = x.reshape(B).astype(jnp.int32)
    out_flat = _gather_scale(table, idx_flat)
    return out_flat.reshape(BATCH, HIST, EMBED)


# speedup vs baseline: 1.2855x; 1.2855x over previous
import math

import jax
import jax.numpy as jnp
from jax.experimental import pallas as pl
from jax.experimental.pallas import tpu as pltpu
from jax.experimental.pallas import tpu_sc as plsc

VOCAB = 1000000
EMBED = 64
BATCH = 16384
HIST = 50
B = BATCH * HIST

WINDOW = 128
LANES = 16
SCALE = math.sqrt(EMBED)

_mesh = plsc.VectorSubcoreMesh(core_axis_name="core", subcore_axis_name="subcore")


def _gather_scale(table, idx_flat):
    idx2d = idx_flat.reshape(1, B)

    @pl.kernel(
        out_type=jax.ShapeDtypeStruct((B, EMBED), jnp.float32),
        mesh=_mesh,
        scratch_types=[pltpu.VMEM((WINDOW, EMBED), jnp.float32)],
        compiler_params=pltpu.CompilerParams(use_tc_tiling_on_sc=False),
    )
    def kern(table_hbm, i_hbm, o_hbm, rows_v):
        def body(i_vmem, o_vmem):
            pltpu.sync_copy(table_hbm.at[i_vmem.at[0]], rows_v)

            @plsc.parallel_loop(0, WINDOW, unroll=8)
            def _(r):
                for c in range(0, EMBED, LANES):
                    o_vmem[r, pl.ds(c, LANES)] = (
                        rows_v[r, pl.ds(c, LANES)] * SCALE
                    )

        pltpu.emit_pipeline(
            body,
            grid=(B // WINDOW,),
            in_specs=[pl.BlockSpec((1, WINDOW), index_map=lambda i: (0, i))],
            out_specs=[pl.BlockSpec((WINDOW, EMBED), index_map=lambda i: (i, 0))],
            core_axis_name=("core", "subcore"),
            dimension_semantics=(pltpu.PARALLEL,),
        )(i_hbm, o_hbm)

    return kern(table, idx2d)


@jax.jit
def kernel(x, table):
    idx_flat = x.reshape(B).astype(jnp.int32)
    out_flat = _gather_scale(table, idx_flat)
    return out_flat.reshape(BATCH, HIST, EMBED)


# speedup vs baseline: 1.3679x; 1.0641x over previous
import math

import jax
import jax.numpy as jnp
from jax.experimental import pallas as pl
from jax.experimental.pallas import tpu as pltpu
from jax.experimental.pallas import tpu_sc as plsc

VOCAB = 1000000
EMBED = 64
BATCH = 16384
HIST = 50

R = 8
LANES = 16
SCALE = math.sqrt(EMBED)

_mesh = plsc.VectorSubcoreMesh(core_axis_name="core", subcore_axis_name="subcore")


def _lookup(table, x):
    @pl.kernel(
        out_type=jax.ShapeDtypeStruct((BATCH, HIST, EMBED), jnp.float32),
        mesh=_mesh,
        scratch_types=[
            pltpu.VMEM((R, HIST, EMBED), jnp.float32),
            pltpu.SemaphoreType.DMA,
        ],
        compiler_params=pltpu.CompilerParams(use_tc_tiling_on_sc=False),
    )
    def kern(table_hbm, x_hbm, o_hbm, rows_v, sem):
        def body(i_vmem, o_vmem):
            copies = [
                pltpu.async_copy(table_hbm.at[i_vmem.at[r]], rows_v.at[r], sem)
                for r in range(R)
            ]
            for c in copies:
                c.wait()

            for r in range(R):
                @plsc.parallel_loop(0, HIST, unroll=5)
                def _(h, r=r):
                    for c in range(0, EMBED, LANES):
                        o_vmem[r, h, pl.ds(c, LANES)] = (
                            rows_v[r, h, pl.ds(c, LANES)] * SCALE
                        )

        pltpu.emit_pipeline(
            body,
            grid=(BATCH // R,),
            in_specs=[pl.BlockSpec((R, HIST), index_map=lambda i: (i, 0))],
            out_specs=[
                pl.BlockSpec((R, HIST, EMBED), index_map=lambda i: (i, 0, 0))
            ],
            core_axis_name=("core", "subcore"),
            dimension_semantics=(pltpu.PARALLEL,),
        )(x_hbm, o_hbm)

    return kern(table, x)


@jax.jit
def kernel(x, table):
    return _lookup(table, x.astype(jnp.int32))
